# Initial kernel scaffold; baseline (speedup 1.0000x reference)
import jax
import jax.numpy as jnp
from jax.experimental import pallas as pl


def kernel(X, A, W1, b1, W2, b2, W3, b3):
    raise NotImplementedError("write your pallas kernel here")


# speedup vs baseline: 6.4150x; 6.4150x over previous
import functools

import jax
import jax.numpy as jnp
from jax import lax
from jax.experimental import pallas as pl
from jax.experimental.pallas import tpu as pltpu
from jax.experimental.pallas import tpu_sc as plsc

N = 10000
E = 320000
D_IN = 128
D_HID = 256

NP = 10112
ROWS_PER_TILE = NP // 16
B = 128
NB16 = 160
NB32 = 80
NBG = NB16 // 8
EPAD = 16 * NB16 * B

_CHUNKS = [(0, 128), (128, 128), (256, 128), (384, 128), (512, 120)]

_MESH = plsc.VectorSubcoreMesh(core_axis_name="c", subcore_axis_name="s")


@functools.partial(
    pl.kernel,
    out_type=jax.ShapeDtypeStruct((2, NP, 128), jnp.float32),
    mesh=_MESH,
    scratch_types=[
        pltpu.VMEM_SHARED((NP, 128), jnp.float32),
        pltpu.VMEM((8, B), jnp.int32),
        pltpu.VMEM((B, 128), jnp.float32),
    ],
)
def _sc_degree(dst32, zrow, orow, deg_out, deg_sh, dstv, obuf):
    c = lax.axis_index("c")
    s = lax.axis_index("s")
    w = c * 16 + s
    pltpu.sync_copy(zrow, obuf)
    base = s * ROWS_PER_TILE
    for off, ln in _CHUNKS:
        pltpu.sync_copy(obuf.at[pl.ds(0, ln)], deg_sh.at[pl.ds(base + off, ln)])
    plsc.subcore_barrier()
    pltpu.sync_copy(orow, obuf)

    def body(g, carry):
        pltpu.sync_copy(dst32.at[w].at[pl.ds(g * 8, 8)], dstv)
        for b in range(8):
            pltpu.sync_copy(obuf, deg_sh.at[dstv.at[b]], add=True)
        return carry

    lax.fori_loop(0, NB32 // 8, body, 0)
    plsc.subcore_barrier()
    for off, ln in _CHUNKS:
        pltpu.sync_copy(deg_sh.at[pl.ds(base + off, ln)], obuf.at[pl.ds(0, ln)])
        pltpu.sync_copy(obuf.at[pl.ds(0, ln)], deg_out.at[c].at[pl.ds(base + off, ln)])


@functools.partial(
    pl.kernel,
    out_type=jax.ShapeDtypeStruct((2, NP, 128), jnp.float32),
    mesh=_MESH,
    scratch_types=[
        pltpu.VMEM_SHARED((NP, 128), jnp.float32),
        pltpu.VMEM((8, B), jnp.int32),
        pltpu.VMEM((8, B), jnp.int32),
        pltpu.VMEM((B, 128), jnp.float32),
        pltpu.SemaphoreType.DMA,
    ],
)
def _sc_scatter(y, src16, dst16, zrow, acc_out,
                acc_sh, srcv, dstv, gbuf, sem):
    c = lax.axis_index("c")
    s = lax.axis_index("s")
    pltpu.sync_copy(zrow, gbuf)
    base = s * ROWS_PER_TILE
    for off, ln in _CHUNKS:
        pltpu.sync_copy(gbuf.at[pl.ds(0, ln)], acc_sh.at[pl.ds(base + off, ln)])
    plsc.subcore_barrier()

    def body(g, carry):
        pltpu.sync_copy(src16.at[s].at[pl.ds(g * 8, 8)], srcv)
        pltpu.sync_copy(dst16.at[s].at[pl.ds(g * 8, 8)], dstv)
        for b in range(8):
            pltpu.async_copy(y.at[c].at[srcv.at[b]], gbuf, sem).wait()
            pltpu.sync_copy(gbuf, acc_sh.at[dstv.at[b]], add=True)
        return carry

    lax.fori_loop(0, NBG, body, 0)
    plsc.subcore_barrier()
    for off, ln in _CHUNKS:
        pltpu.sync_copy(acc_sh.at[pl.ds(base + off, ln)], gbuf.at[pl.ds(0, ln)])
        pltpu.sync_copy(gbuf.at[pl.ds(0, ln)], acc_out.at[c].at[pl.ds(base + off, ln)])


_R = 1000


def _dinv_from(deg_ref):
    dtot = 1.0 + deg_ref[0, :, 0] + deg_ref[1, :, 0]
    return lax.rsqrt(dtot)[:, None]


def _k_first(x_ref, w_ref, deg_ref, y_ref):
    dinv = _dinv_from(deg_ref)
    y = jnp.dot(x_ref[...], w_ref[...], preferred_element_type=jnp.float32) * dinv
    y_ref[0] = y[:, :128]
    y_ref[1] = y[:, 128:]


def _k_mid(acc_ref, y_ref, deg_ref, b_ref, w_ref, o_ref):
    dinv = _dinv_from(deg_ref)
    cat = jnp.concatenate(
        [acc_ref[0] + y_ref[0], acc_ref[1] + y_ref[1]], axis=1)
    h = jnp.maximum(cat * dinv + b_ref[0], 0.0)
    y2 = jnp.dot(h, w_ref[...], preferred_element_type=jnp.float32) * dinv
    o_ref[0] = y2[:, :128]
    o_ref[1] = y2[:, 128:]


def _k_last(acc_ref, y_ref, deg_ref, b_ref, o_ref):
    dinv = _dinv_from(deg_ref)
    cat = jnp.concatenate(
        [acc_ref[0] + y_ref[0], acc_ref[1] + y_ref[1]], axis=1)
    o_ref[...] = cat * dinv + b_ref[0]


def _tc_first(x, w, deg2):
    return pl.pallas_call(
        _k_first,
        grid=(N // _R,),
        in_specs=[
            pl.BlockSpec((_R, D_IN), lambda i: (i, 0)),
            pl.BlockSpec((D_IN, D_HID), lambda i: (0, 0)),
            pl.BlockSpec((2, _R, 128), lambda i: (0, i, 0)),
        ],
        out_specs=pl.BlockSpec((2, _R, 128), lambda i: (0, i, 0)),
        out_shape=jax.ShapeDtypeStruct((2, N, 128), jnp.float32),
    )(x, w, deg2)


def _tc_mid(acc, y, deg2, b, w):
    return pl.pallas_call(
        _k_mid,
        grid=(N // _R,),
        in_specs=[
            pl.BlockSpec((2, _R, 128), lambda i: (0, i, 0)),
            pl.BlockSpec((2, _R, 128), lambda i: (0, i, 0)),
            pl.BlockSpec((2, _R, 128), lambda i: (0, i, 0)),
            pl.BlockSpec((1, D_HID), lambda i: (0, 0)),
            pl.BlockSpec((D_HID, D_HID), lambda i: (0, 0)),
        ],
        out_specs=pl.BlockSpec((2, _R, 128), lambda i: (0, i, 0)),
        out_shape=jax.ShapeDtypeStruct((2, N, 128), jnp.float32),
    )(acc, y, deg2, b, w)


def _tc_last(acc, y, deg2, b):
    return pl.pallas_call(
        _k_last,
        grid=(N // _R,),
        in_specs=[
            pl.BlockSpec((2, _R, 128), lambda i: (0, i, 0)),
            pl.BlockSpec((2, _R, 128), lambda i: (0, i, 0)),
            pl.BlockSpec((2, _R, 128), lambda i: (0, i, 0)),
            pl.BlockSpec((1, D_HID), lambda i: (0, 0)),
        ],
        out_specs=pl.BlockSpec((_R, D_HID), lambda i: (i, 0)),
        out_shape=jax.ShapeDtypeStruct((N, D_HID), jnp.float32),
    )(acc, y, deg2, b)


def kernel(X, A, W1, b1, W2, b2, W3, b3):
    a32 = A.astype(jnp.int32)
    src = jnp.concatenate([a32[0], jnp.zeros((EPAD - E,), jnp.int32)])
    dst = jnp.concatenate([a32[1], jnp.full((EPAD - E,), N, jnp.int32)])
    src16 = src.reshape(16, NB16, B)
    dst16 = dst.reshape(16, NB16, B)
    dst32 = dst.reshape(32, NB32, B)

    zrow = jnp.zeros((B, 128), jnp.float32)
    orow = jnp.ones((B, 128), jnp.float32)

    deg2 = _sc_degree(dst32, zrow, orow)

    y1 = _tc_first(X, W1, deg2)
    acc1 = _sc_scatter(y1, src16, dst16, zrow)
    y2 = _tc_mid(acc1, y1, deg2, b1.reshape(1, -1), W2)
    acc2 = _sc_scatter(y2, src16, dst16, zrow)
    y3 = _tc_mid(acc2, y2, deg2, b2.reshape(1, -1), W3)
    acc3 = _sc_scatter(y3, src16, dst16, zrow)
    return _tc_last(acc3, y3, deg2, b3.reshape(1, -1))
